# baseline (speedup 1.0000x reference)
import functools

import jax
import jax.numpy as jnp
from jax import lax
from jax.experimental import pallas as pl
from jax.experimental.pallas import tpu as pltpu
from jax.experimental.pallas import tpu_sc as plsc

_GATHER_BLOCK = 8
_CHUNK = 2


def _sc_assemble(x, emb, out_sds):
    b, t, d = x.shape
    e = emb.shape[1]
    try:
        info = plsc.get_sparse_core_info()
        num_cores, num_subcores = info.num_cores, info.num_subcores
    except Exception:
        num_cores, num_subcores = 2, 16
    num_workers = num_cores * num_subcores
    b_per_w = b // num_workers
    ch = _CHUNK
    mesh = plsc.VectorSubcoreMesh(core_axis_name="c", subcore_axis_name="s")

    @functools.partial(
        pl.kernel,
        out_type=out_sds,
        mesh=mesh,
        scratch_types=[
            pltpu.VMEM((e,), jnp.float32),
            pltpu.VMEM((t, d), jnp.float32),
            pltpu.VMEM((t, d + e), jnp.float32),
            pltpu.VMEM((t, d + e), jnp.float32),
            pltpu.SemaphoreType.DMA,
            pltpu.SemaphoreType.DMA,
        ],
    )
    def assemble(x_hbm, emb_hbm, out_hbm, stage, xb, comb0, comb1,
                 sem_in, sem_out):
        combs = (comb0, comb1)
        wid = lax.axis_index("s") * num_cores + lax.axis_index("c")
        base = wid * b_per_w

        def slot(u, j):
            row = base + u
            comb = combs[j]

            @pl.when(u >= 2)
            def _():
                pltpu.make_async_copy(comb, out_hbm.at[row - 2], sem_out).wait()

            cp_in = pltpu.async_copy(x_hbm.at[row], xb, sem_in)
            pltpu.sync_copy(emb_hbm.at[row], stage)
            evs = [stage[pl.ds(16 * c, 16)] for c in range(e // 16)]
            cp_in.wait()

            def tbody(tt, carry):
                for c in range(d // 16):
                    comb[tt, pl.ds(16 * c, 16)] = xb[tt, pl.ds(16 * c, 16)]
                for c in range(e // 16):
                    comb[tt, pl.ds(d + 16 * c, 16)] = evs[c]
                return carry

            lax.fori_loop(0, t, tbody, 0)
            pltpu.async_copy(comb, out_hbm.at[row], sem_out)

        def g_body(g, carry):
            for j in range(2):
                slot(g * 2 + j, j)
            return carry

        lax.fori_loop(0, b_per_w // 2, g_body, 0)
        for j in range(2):
            u = b_per_w - 2 + j
            pltpu.make_async_copy(
                combs[j], out_hbm.at[base + u], sem_out).wait()

    return assemble(x, emb)


def _gather_body(sid_ref, *refs):
    g = _GATHER_BLOCK
    table_refs, emb_ref = refs[:-1], refs[-1]
    i = pl.program_id(0)
    for k, tref in enumerate(table_refs):
        r = sid_ref[i * g + k] % 8
        emb_ref[k, :] = tref[pl.ds(r, 1), :][0, :]


def kernel(x, speaker_id, emb_table):
    b, t, d = x.shape
    e = emb_table.shape[1]
    g = _GATHER_BLOCK
    out_sds = jax.ShapeDtypeStruct((b, t, d + e), jnp.float32)

    emb = pl.pallas_call(
        _gather_body,
        grid_spec=pltpu.PrefetchScalarGridSpec(
            num_scalar_prefetch=1,
            grid=(b // g,),
            in_specs=[
                pl.BlockSpec((8, e), functools.partial(
                    lambda k, i, sid: (sid[i * g + k] // 8, 0), k))
                for k in range(g)
            ],
            out_specs=pl.BlockSpec((g, e), lambda i, sid: (i, 0)),
        ),
        out_shape=jax.ShapeDtypeStruct((b, e), jnp.float32),
    )(speaker_id.astype(jnp.int32), *([emb_table] * g))

    return _sc_assemble(x, emb, out_sds)


# speedup vs baseline: 1.3370x; 1.3370x over previous
import functools

import jax
import jax.numpy as jnp
from jax import lax
from jax.experimental import pallas as pl
from jax.experimental.pallas import tpu as pltpu
from jax.experimental.pallas import tpu_sc as plsc

_GATHER_BLOCK = 8
_CHUNK = 2


def _sc_assemble(x, emb, out_sds):
    b, t, d = x.shape
    e = emb.shape[1]
    try:
        info = plsc.get_sparse_core_info()
        num_cores, num_subcores = info.num_cores, info.num_subcores
    except Exception:
        num_cores, num_subcores = 2, 16
    num_workers = num_cores * num_subcores
    b_per_w = b // num_workers
    ch = _CHUNK
    mesh = plsc.VectorSubcoreMesh(core_axis_name="c", subcore_axis_name="s")

    ntg = t // 8
    nxb = 2

    @functools.partial(
        pl.kernel,
        out_type=out_sds,
        mesh=mesh,
        scratch_types=[
            pltpu.VMEM((b_per_w, e), jnp.float32),
            *[pltpu.VMEM((t, e), jnp.float32) for _ in range(2)],
            *[pltpu.VMEM((b_per_w, 8, d), jnp.float32) for _ in range(nxb)],
            pltpu.SemaphoreType.DMA,
            pltpu.SemaphoreType.DMA,
            pltpu.SemaphoreType.DMA,
        ],
    )
    def assemble(x_hbm, emb_hbm, out_hbm, emb_v, bc0, bc1, *rest):
        xbs = rest[:nxb]
        sem_in, sem_out, sem_e = rest[nxb:]
        bcs = (bc0, bc1)
        wid = lax.axis_index("s") * num_cores + lax.axis_index("c")
        base = wid * b_per_w
        pltpu.sync_copy(emb_hbm.at[pl.ds(base, b_per_w)], emb_v)

        def in_cp(tg, j):
            return pltpu.async_copy(
                x_hbm.at[pl.ds(base, b_per_w), pl.ds(8 * tg, 8), :],
                xbs[j], sem_in)

        def outx_cp(tg, j):
            return pltpu.async_copy(
                xbs[j],
                out_hbm.at[pl.ds(base, b_per_w), pl.ds(8 * tg, 8),
                           pl.ds(0, d)],
                sem_out)

        cps_in = {}
        for tg in range(min(nxb, ntg)):
            cps_in[tg] = in_cp(tg, tg % nxb)
        outs_x = {}
        for tg in range(ntg):
            j = tg % nxb
            cps_in[tg].wait()
            outs_x[tg] = outx_cp(tg, j)
            if tg + nxb < ntg:
                outs_x.pop(tg).wait()
                cps_in[tg + nxb] = in_cp(tg + nxb, j)

        def erow(r, j):
            bc = bcs[j]

            @pl.when(r >= 2)
            def _():
                pltpu.make_async_copy(
                    bc, out_hbm.at[base + r - 2, :, pl.ds(d, e)],
                    sem_e).wait()

            evs = [emb_v[r, pl.ds(16 * c, 16)] for c in range(e // 16)]
            for tt in range(t):
                for c in range(e // 16):
                    bc[tt, pl.ds(16 * c, 16)] = evs[c]
            pltpu.async_copy(bc, out_hbm.at[base + r, :, pl.ds(d, e)], sem_e)

        def e_body(g, carry):
            for j in range(2):
                erow(g * 2 + j, j)
            return carry

        lax.fori_loop(0, b_per_w // 2, e_body, 0)
        for j in range(2):
            r = b_per_w - 2 + j
            pltpu.make_async_copy(
                bcs[j], out_hbm.at[base + r, :, pl.ds(d, e)], sem_e).wait()
        for cp in outs_x.values():
            cp.wait()

    return assemble(x, emb)


def _gather_body(sid_ref, *refs):
    g = _GATHER_BLOCK
    table_refs, emb_ref = refs[:-1], refs[-1]
    i = pl.program_id(0)
    for k, tref in enumerate(table_refs):
        r = sid_ref[i * g + k] % 8
        emb_ref[k, :] = tref[pl.ds(r, 1), :][0, :]


def kernel(x, speaker_id, emb_table):
    b, t, d = x.shape
    e = emb_table.shape[1]
    g = _GATHER_BLOCK
    out_sds = jax.ShapeDtypeStruct((b, t, d + e), jnp.float32)

    emb = pl.pallas_call(
        _gather_body,
        grid_spec=pltpu.PrefetchScalarGridSpec(
            num_scalar_prefetch=1,
            grid=(b // g,),
            in_specs=[
                pl.BlockSpec((8, e), functools.partial(
                    lambda k, i, sid: (sid[i * g + k] // 8, 0), k))
                for k in range(g)
            ],
            out_specs=pl.BlockSpec((g, e), lambda i, sid: (i, 0)),
        ),
        out_shape=jax.ShapeDtypeStruct((b, e), jnp.float32),
    )(speaker_id.astype(jnp.int32), *([emb_table] * g))

    return _sc_assemble(x, emb, out_sds)


# speedup vs baseline: 1.5133x; 1.1319x over previous
import functools

import jax
import jax.numpy as jnp
from jax import lax
from jax.experimental import pallas as pl
from jax.experimental.pallas import tpu as pltpu
from jax.experimental.pallas import tpu_sc as plsc


def _sc_gather(emb_table, speaker_id):
    n_rows, emb_dim = emb_table.shape
    batch = speaker_id.shape[0]
    try:
        info = plsc.get_sparse_core_info()
        num_cores, num_subcores = info.num_cores, info.num_subcores
    except Exception:
        num_cores, num_subcores = 2, 16
    num_workers = num_cores * num_subcores
    b_per_w = batch // num_workers
    mesh = plsc.VectorSubcoreMesh(core_axis_name="c", subcore_axis_name="s")

    @functools.partial(
        pl.kernel,
        out_type=jax.ShapeDtypeStruct((batch, emb_dim), jnp.float32),
        mesh=mesh,
        compiler_params=pltpu.CompilerParams(use_tc_tiling_on_sc=False),
        scratch_types=[
            pltpu.VMEM((b_per_w,), jnp.int32),
            pltpu.VMEM((b_per_w, emb_dim), jnp.float32),
            pltpu.SemaphoreType.DMA,
        ],
    )
    def gather_kernel(table_hbm, idx_hbm, out_hbm, idx_v, rows_v, sem):
        wid = lax.axis_index("s") * num_cores + lax.axis_index("c")
        base = wid * b_per_w
        pltpu.sync_copy(idx_hbm.at[pl.ds(base, b_per_w)], idx_v)
        pltpu.async_copy(table_hbm.at[idx_v], rows_v, sem).wait()
        pltpu.sync_copy(rows_v, out_hbm.at[pl.ds(base, b_per_w)])

    return gather_kernel(emb_table, speaker_id)


def _concat_body(x_ref, emb_ref, o_ref):
    bb, t, d = x_ref.shape
    e = emb_ref.shape[-1]
    o_ref[:, :, :d] = x_ref[...]
    emb = emb_ref[...]
    o_ref[:, :, d:] = jnp.broadcast_to(emb[:, None, :], (bb, t, e))


def kernel(x, speaker_id, emb_table):
    b, t, d = x.shape
    e = emb_table.shape[1]
    emb = _sc_gather(emb_table, speaker_id.astype(jnp.int32))
    bb = 32
    return pl.pallas_call(
        _concat_body,
        grid=(b // bb,),
        in_specs=[
            pl.BlockSpec((bb, t, d), lambda i: (i, 0, 0)),
            pl.BlockSpec((bb, e), lambda i: (i, 0)),
        ],
        out_specs=pl.BlockSpec((bb, t, d + e), lambda i: (i, 0, 0)),
        out_shape=jax.ShapeDtypeStruct((b, t, d + e), jnp.float32),
    )(x, emb)


# speedup vs baseline: 1.5227x; 1.0062x over previous
import functools

import jax
import jax.numpy as jnp
from jax import lax
from jax.experimental import pallas as pl
from jax.experimental.pallas import tpu as pltpu
from jax.experimental.pallas import tpu_sc as plsc


def _sc_gather(emb_table, speaker_id):
    n_rows, emb_dim = emb_table.shape
    batch = speaker_id.shape[0]
    try:
        info = plsc.get_sparse_core_info()
        num_cores, num_subcores = info.num_cores, info.num_subcores
    except Exception:
        num_cores, num_subcores = 2, 16
    num_workers = num_cores * num_subcores
    b_per_w = batch // num_workers
    mesh = plsc.VectorSubcoreMesh(core_axis_name="c", subcore_axis_name="s")

    @functools.partial(
        pl.kernel,
        out_type=jax.ShapeDtypeStruct((batch, emb_dim), jnp.float32),
        mesh=mesh,
        compiler_params=pltpu.CompilerParams(use_tc_tiling_on_sc=False),
        scratch_types=[
            pltpu.VMEM((b_per_w,), jnp.int32),
            pltpu.VMEM((b_per_w, emb_dim), jnp.float32),
            pltpu.SemaphoreType.DMA,
        ],
    )
    def gather_kernel(table_hbm, idx_hbm, out_hbm, idx_v, rows_v, sem):
        wid = lax.axis_index("s") * num_cores + lax.axis_index("c")
        base = wid * b_per_w
        pltpu.sync_copy(idx_hbm.at[pl.ds(base, b_per_w)], idx_v)
        pltpu.async_copy(table_hbm.at[idx_v], rows_v, sem).wait()
        pltpu.sync_copy(rows_v, out_hbm.at[pl.ds(base, b_per_w)])

    return gather_kernel(emb_table, speaker_id)


def _concat_body(x_ref, emb_ref, o_ref):
    bb, t, d = x_ref.shape
    e = emb_ref.shape[-1]
    o_ref[:, :, :d] = x_ref[...]
    emb = emb_ref[...]
    o_ref[:, :, d:] = jnp.broadcast_to(emb[:, None, :], (bb, t, e))


def kernel(x, speaker_id, emb_table):
    b, t, d = x.shape
    e = emb_table.shape[1]
    emb = _sc_gather(emb_table, speaker_id.astype(jnp.int32))
    bb = 64
    return pl.pallas_call(
        _concat_body,
        grid=(b // bb,),
        in_specs=[
            pl.BlockSpec((bb, t, d), lambda i: (i, 0, 0)),
            pl.BlockSpec((bb, e), lambda i: (i, 0)),
        ],
        out_specs=pl.BlockSpec((bb, t, d + e), lambda i: (i, 0, 0)),
        out_shape=jax.ShapeDtypeStruct((b, t, d + e), jnp.float32),
    )(x, emb)
